# baseline (speedup 1.0000x reference)
import functools

import jax
import jax.numpy as jnp
from jax import lax
from jax.experimental import pallas as pl
from jax.experimental.pallas import tpu as pltpu
from jax.experimental.pallas import tpu_sc as plsc

B = 8
N = 2048
D = 3
K = 16
QB = 128
NS = 640
NT = N - NS
NW = 32
WPB = NW // B
QS = NS // WPB
L = 16

NEG_SCALE = 1e-33
NEG_OFF = 0.125
BIGF = 3e38
IMASK = ~(N - 1)


def _knn_tile(q_ref, pt_ref, dist_ref, src_ref, dst_ref):
    b = pl.program_id(0)
    qb = pl.program_id(1)

    q = q_ref[0]
    p = pt_ref[0]

    dots = jax.lax.dot_general(
        q.astype(jnp.bfloat16), p.astype(jnp.bfloat16),
        (((1,), (0,)), ((), ())),
        preferred_element_type=jnp.float32,
    )
    x2q = jnp.sum(q * q, axis=1, keepdims=True)
    x2p = jnp.sum(p * p, axis=0, keepdims=True)
    dist = x2q + x2p - 2.0 * dots

    cols = jax.lax.broadcasted_iota(jnp.int32, (QB, N), 1)
    rows = jax.lax.broadcasted_iota(jnp.int32, (QB, 1), 0)
    q_glob = rows + qb * QB

    dk = jnp.where(dist < 0.0, (dist + NEG_OFF) * NEG_SCALE, dist)
    keys = ((dk.view(jnp.int32) & IMASK) | cols).view(jnp.float32)
    keys = jnp.where(cols == q_glob, BIGF, keys)

    offset = b * N
    ms = []
    for _ in range(K):
        m = jnp.min(keys, axis=1, keepdims=True)
        ms.append(m)
        keys = jnp.where(keys == m, BIGF, keys)

    mk = jnp.concatenate(ms, axis=1).view(jnp.int32)
    idx = mk & jnp.int32(N - 1)
    val = (mk & jnp.int32(~(N - 1))).view(jnp.float32)
    val = jnp.where((val > 0.0) & (val < jnp.float32(1e-30)),
                    val / NEG_SCALE - NEG_OFF, val)

    dist_ref[0] = val
    src_ref[0] = idx + offset
    dst_ref[0] = jnp.broadcast_to(q_glob + offset, (QB, K))


def _bf16_round(v):
    bits = lax.bitcast_convert_type(v, jnp.int32)
    rbits = (bits + 0x7FFF + (lax.shift_right_logical(bits, 16) & 1)) & ~0xFFFF
    return lax.bitcast_convert_type(rbits, jnp.float32)


def _sc_knn(pts_ref, outd_ref, outs_ref, pf, pb, x2b, resd, ress):
    wid = lax.axis_index("s") * 2 + lax.axis_index("c")
    b = wid // WPB
    part = wid % WPB
    qstart = NT + part * QS

    pltpu.sync_copy(pts_ref.at[b], pf)

    lane = lax.iota(jnp.int32, L)

    def stage(c, _):
        sl = pl.ds(c * L, L)
        vx = pf[0, sl]
        vy = pf[1, sl]
        vz = pf[2, sl]
        pb[0, sl] = _bf16_round(vx)
        pb[1, sl] = _bf16_round(vy)
        pb[2, sl] = _bf16_round(vz)
        x2b[sl] = vx * vx + vy * vy + vz * vz
        return 0

    lax.fori_loop(0, N // L, stage, 0)

    def per_query(qi, _):
        q = qstart + qi
        qsl = pl.ds((q // L) * L, L)
        lsel = jnp.full((L,), q % L, jnp.int32)

        def bcast(v):
            return v.at[lsel].get(mode="promise_in_bounds")

        qx = bcast(pf[0, qsl])
        qy = bcast(pf[1, qsl])
        qz = bcast(pf[2, qsl])
        x2q = bcast(x2b[qsl])
        qxb = _bf16_round(qx)
        qyb = _bf16_round(qy)
        qzb = _bf16_round(qz)

        def chunk(c, state):
            sl = pl.ds(c * L, L)
            dots = qxb * pb[0, sl]
            dots = dots + qyb * pb[1, sl]
            dots = dots + qzb * pb[2, sl]
            dist = (x2q + x2b[sl]) - 2.0 * dots
            dk = jnp.where(dist < 0.0, (dist + NEG_OFF) * NEG_SCALE, dist)
            col = lane + c * L
            keys = lax.bitcast_convert_type(
                (lax.bitcast_convert_type(dk, jnp.int32) & IMASK) | col,
                jnp.float32)
            v = jnp.where(col == q, BIGF, keys)
            out = []
            for s in state:
                lo = jnp.minimum(s, v)
                v = jnp.maximum(s, v)
                out.append(lo)
            return tuple(out)

        init = tuple(jnp.full((L,), BIGF, jnp.float32) for _ in range(K))
        state = lax.fori_loop(0, N // L, chunk, init, unroll=4)

        cur = list(state)
        res = jnp.full((L,), BIGF, jnp.float32)
        for k in range(K):
            m = cur[0]
            for s in cur[1:]:
                m = jnp.minimum(m, s)
            for h in (8, 4, 2, 1):
                m = jnp.minimum(m, m.at[lane ^ h].get(mode="promise_in_bounds"))
            res = jnp.where(lane == k, m, res)
            cur = [jnp.where(s == m, BIGF, s) for s in cur]

        kb = lax.bitcast_convert_type(res, jnp.int32)
        idx = kb & (N - 1)
        vb = lax.bitcast_convert_type(kb & IMASK, jnp.float32)
        val = jnp.where((vb > 0.0) & (vb < 1e-30),
                        vb / NEG_SCALE - NEG_OFF, vb)
        osl = pl.ds(qi * K, K)
        resd[osl] = val
        ress[osl] = idx + b * N
        return 0

    lax.fori_loop(0, QS, per_query, 0)

    off = (b * NS + part * QS) * K
    pltpu.sync_copy(resd, outd_ref.at[pl.ds(off, QS * K)])
    pltpu.sync_copy(ress, outs_ref.at[pl.ds(off, QS * K)])


_sc_call = functools.partial(
    pl.kernel,
    out_type=[
        jax.ShapeDtypeStruct((B * NS * K,), jnp.float32),
        jax.ShapeDtypeStruct((B * NS * K,), jnp.int32),
    ],
    mesh=plsc.VectorSubcoreMesh(core_axis_name="c", subcore_axis_name="s"),
    scratch_types=[
        pltpu.VMEM((D, N), jnp.float32),
        pltpu.VMEM((D, N), jnp.float32),
        pltpu.VMEM((N,), jnp.float32),
        pltpu.VMEM((QS * K,), jnp.float32),
        pltpu.VMEM((QS * K,), jnp.int32),
    ],
)(_sc_knn)


@jax.jit
def kernel(points):
    pts_t = jnp.transpose(points, (0, 2, 1))
    grid = (B, NT // QB)
    dist_tc, src_tc, dst_tc = pl.pallas_call(
        _knn_tile,
        grid=grid,
        in_specs=[
            pl.BlockSpec((1, QB, D), lambda b, q: (b, q, 0)),
            pl.BlockSpec((1, D, N), lambda b, q: (b, 0, 0)),
        ],
        out_specs=[
            pl.BlockSpec((1, QB, K), lambda b, q: (b, q, 0)),
            pl.BlockSpec((1, QB, K), lambda b, q: (b, q, 0)),
            pl.BlockSpec((1, QB, K), lambda b, q: (b, q, 0)),
        ],
        out_shape=[
            jax.ShapeDtypeStruct((B, NT, K), jnp.float32),
            jax.ShapeDtypeStruct((B, NT, K), jnp.int32),
            jax.ShapeDtypeStruct((B, NT, K), jnp.int32),
        ],
    )(points, pts_t)

    dist_sc, src_sc = _sc_call(pts_t)
    dist_sc = dist_sc.reshape(B, NS, K)
    src_sc = src_sc.reshape(B, NS, K)
    dst_sc = (jnp.arange(NT, N, dtype=jnp.int32)[None, :, None]
              + (jnp.arange(B, dtype=jnp.int32) * N)[:, None, None])
    dst_sc = jnp.broadcast_to(dst_sc, (B, NS, K))

    dist = jnp.concatenate([dist_tc, dist_sc], axis=1)
    src = jnp.concatenate([src_tc, src_sc], axis=1)
    dst = jnp.concatenate([dst_tc, dst_sc], axis=1)
    edge_index = jnp.stack([src.reshape(-1), dst.reshape(-1)], axis=0)
    return (dist, edge_index)


# speedup vs baseline: 1.0177x; 1.0177x over previous
import functools

import jax
import jax.numpy as jnp
from jax import lax
from jax.experimental import pallas as pl
from jax.experimental.pallas import tpu as pltpu
from jax.experimental.pallas import tpu_sc as plsc

B = 8
N = 2048
D = 3
K = 16
QB = 256
NS = 512
NT = N - NS
NW = 32
WPB = NW // B
QS = NS // WPB
L = 16

NEG_SCALE = 1e-33
NEG_OFF = 0.125
BIGF = 3e38
IMASK = ~(N - 1)


def _knn_tile(q_ref, pt_ref, dist_ref, src_ref, dst_ref):
    b = pl.program_id(0)
    qb = pl.program_id(1)

    q = q_ref[0]
    p = pt_ref[0]

    dots = jax.lax.dot_general(
        q.astype(jnp.bfloat16), p.astype(jnp.bfloat16),
        (((1,), (0,)), ((), ())),
        preferred_element_type=jnp.float32,
    )
    x2q = jnp.sum(q * q, axis=1, keepdims=True)
    x2p = jnp.sum(p * p, axis=0, keepdims=True)
    dist = x2q + x2p - 2.0 * dots

    cols = jax.lax.broadcasted_iota(jnp.int32, (QB, N), 1)
    rows = jax.lax.broadcasted_iota(jnp.int32, (QB, 1), 0)
    q_glob = rows + qb * QB

    dk = jnp.where(dist < 0.0, (dist + NEG_OFF) * NEG_SCALE, dist)
    keys = ((dk.view(jnp.int32) & IMASK) | cols).view(jnp.float32)
    keys = jnp.where(cols == q_glob, BIGF, keys)

    offset = b * N
    ms = []
    for _ in range(K):
        m = jnp.min(keys, axis=1, keepdims=True)
        ms.append(m)
        keys = jnp.where(keys == m, BIGF, keys)

    mk = jnp.concatenate(ms, axis=1).view(jnp.int32)
    idx = mk & jnp.int32(N - 1)
    val = (mk & jnp.int32(~(N - 1))).view(jnp.float32)
    val = jnp.where((val > 0.0) & (val < jnp.float32(1e-30)),
                    val / NEG_SCALE - NEG_OFF, val)

    dist_ref[0] = val
    src_ref[0] = idx + offset
    dst_ref[0] = jnp.broadcast_to(q_glob + offset, (QB, K))


def _bf16_round(v):
    bits = lax.bitcast_convert_type(v, jnp.int32)
    rbits = (bits + 0x7FFF + (lax.shift_right_logical(bits, 16) & 1)) & ~0xFFFF
    return lax.bitcast_convert_type(rbits, jnp.float32)


def _sc_knn(pts_ref, outd_ref, outs_ref, pf, pb, x2b, resd, ress):
    wid = lax.axis_index("s") * 2 + lax.axis_index("c")
    b = wid // WPB
    part = wid % WPB
    qstart = NT + part * QS

    pltpu.sync_copy(pts_ref.at[b], pf)

    lane = lax.iota(jnp.int32, L)

    def stage(c, _):
        sl = pl.ds(c * L, L)
        vx = pf[0, sl]
        vy = pf[1, sl]
        vz = pf[2, sl]
        pb[0, sl] = _bf16_round(vx)
        pb[1, sl] = _bf16_round(vy)
        pb[2, sl] = _bf16_round(vz)
        x2b[sl] = vx * vx + vy * vy + vz * vz
        return 0

    lax.fori_loop(0, N // L, stage, 0)

    def per_query(qi, _):
        q = qstart + qi
        qsl = pl.ds((q // L) * L, L)
        lsel = jnp.full((L,), q % L, jnp.int32)

        def bcast(v):
            return v.at[lsel].get(mode="promise_in_bounds")

        qx = bcast(pf[0, qsl])
        qy = bcast(pf[1, qsl])
        qz = bcast(pf[2, qsl])
        x2q = bcast(x2b[qsl])
        qxb = _bf16_round(qx)
        qyb = _bf16_round(qy)
        qzb = _bf16_round(qz)

        def chunk(c, state):
            sl = pl.ds(c * L, L)
            dots = qxb * pb[0, sl]
            dots = dots + qyb * pb[1, sl]
            dots = dots + qzb * pb[2, sl]
            dist = (x2q + x2b[sl]) - 2.0 * dots
            dk = jnp.where(dist < 0.0, (dist + NEG_OFF) * NEG_SCALE, dist)
            col = lane + c * L
            keys = lax.bitcast_convert_type(
                (lax.bitcast_convert_type(dk, jnp.int32) & IMASK) | col,
                jnp.float32)
            v = jnp.where(col == q, BIGF, keys)
            out = []
            for s in state:
                lo = jnp.minimum(s, v)
                v = jnp.maximum(s, v)
                out.append(lo)
            return tuple(out)

        init = tuple(jnp.full((L,), BIGF, jnp.float32) for _ in range(K))
        state = lax.fori_loop(0, N // L, chunk, init)

        cur = list(state)
        res = jnp.full((L,), BIGF, jnp.float32)
        for k in range(K):
            m = cur[0]
            for s in cur[1:]:
                m = jnp.minimum(m, s)
            for h in (8, 4, 2, 1):
                m = jnp.minimum(m, m.at[lane ^ h].get(mode="promise_in_bounds"))
            res = jnp.where(lane == k, m, res)
            cur = [jnp.where(s == m, BIGF, s) for s in cur]

        kb = lax.bitcast_convert_type(res, jnp.int32)
        idx = kb & (N - 1)
        vb = lax.bitcast_convert_type(kb & IMASK, jnp.float32)
        val = jnp.where((vb > 0.0) & (vb < 1e-30),
                        vb / NEG_SCALE - NEG_OFF, vb)
        osl = pl.ds(qi * K, K)
        resd[osl] = val
        ress[osl] = idx + b * N
        return 0

    lax.fori_loop(0, QS, per_query, 0)

    off = (b * NS + part * QS) * K
    pltpu.sync_copy(resd, outd_ref.at[pl.ds(off, QS * K)])
    pltpu.sync_copy(ress, outs_ref.at[pl.ds(off, QS * K)])


_sc_call = functools.partial(
    pl.kernel,
    out_type=[
        jax.ShapeDtypeStruct((B * NS * K,), jnp.float32),
        jax.ShapeDtypeStruct((B * NS * K,), jnp.int32),
    ],
    mesh=plsc.VectorSubcoreMesh(core_axis_name="c", subcore_axis_name="s"),
    scratch_types=[
        pltpu.VMEM((D, N), jnp.float32),
        pltpu.VMEM((D, N), jnp.float32),
        pltpu.VMEM((N,), jnp.float32),
        pltpu.VMEM((QS * K,), jnp.float32),
        pltpu.VMEM((QS * K,), jnp.int32),
    ],
)(_sc_knn)


@jax.jit
def kernel(points):
    pts_t = jnp.transpose(points, (0, 2, 1))
    grid = (B, NT // QB)
    dist_tc, src_tc, dst_tc = pl.pallas_call(
        _knn_tile,
        grid=grid,
        in_specs=[
            pl.BlockSpec((1, QB, D), lambda b, q: (b, q, 0)),
            pl.BlockSpec((1, D, N), lambda b, q: (b, 0, 0)),
        ],
        out_specs=[
            pl.BlockSpec((1, QB, K), lambda b, q: (b, q, 0)),
            pl.BlockSpec((1, QB, K), lambda b, q: (b, q, 0)),
            pl.BlockSpec((1, QB, K), lambda b, q: (b, q, 0)),
        ],
        out_shape=[
            jax.ShapeDtypeStruct((B, NT, K), jnp.float32),
            jax.ShapeDtypeStruct((B, NT, K), jnp.int32),
            jax.ShapeDtypeStruct((B, NT, K), jnp.int32),
        ],
    )(points, pts_t)

    dist_sc, src_sc = _sc_call(pts_t)
    dist_sc = dist_sc.reshape(B, NS, K)
    src_sc = src_sc.reshape(B, NS, K)
    dst_sc = (jnp.arange(NT, N, dtype=jnp.int32)[None, :, None]
              + (jnp.arange(B, dtype=jnp.int32) * N)[:, None, None])
    dst_sc = jnp.broadcast_to(dst_sc, (B, NS, K))

    dist = jnp.concatenate([dist_tc, dist_sc], axis=1)
    src = jnp.concatenate([src_tc, src_sc], axis=1)
    dst = jnp.concatenate([dst_tc, dst_sc], axis=1)
    edge_index = jnp.stack([src.reshape(-1), dst.reshape(-1)], axis=0)
    return (dist, edge_index)


# speedup vs baseline: 1.0203x; 1.0025x over previous
import functools

import jax
import jax.numpy as jnp
from jax import lax
from jax.experimental import pallas as pl
from jax.experimental.pallas import tpu as pltpu
from jax.experimental.pallas import tpu_sc as plsc

B = 8
N = 2048
D = 3
K = 16
QB = 512
NS = 512
NT = N - NS
NW = 32
WPB = NW // B
QS = NS // WPB
L = 16

NEG_SCALE = 1e-33
NEG_OFF = 0.125
BIGF = 3e38
IMASK = ~(N - 1)


def _knn_tile(q_ref, pt_ref, dist_ref, src_ref, dst_ref):
    b = pl.program_id(0)
    qb = pl.program_id(1)

    q = q_ref[0]
    p = pt_ref[0]

    dots = jax.lax.dot_general(
        q.astype(jnp.bfloat16), p.astype(jnp.bfloat16),
        (((1,), (0,)), ((), ())),
        preferred_element_type=jnp.float32,
    )
    x2q = jnp.sum(q * q, axis=1, keepdims=True)
    x2p = jnp.sum(p * p, axis=0, keepdims=True)
    dist = x2q + x2p - 2.0 * dots

    cols = jax.lax.broadcasted_iota(jnp.int32, (QB, N), 1)
    rows = jax.lax.broadcasted_iota(jnp.int32, (QB, 1), 0)
    q_glob = rows + qb * QB

    dk = jnp.where(dist < 0.0, (dist + NEG_OFF) * NEG_SCALE, dist)
    keys = ((dk.view(jnp.int32) & IMASK) | cols).view(jnp.float32)
    keys = jnp.where(cols == q_glob, BIGF, keys)

    offset = b * N
    ms = []
    for _ in range(K):
        m = jnp.min(keys, axis=1, keepdims=True)
        ms.append(m)
        keys = jnp.where(keys == m, BIGF, keys)

    mk = jnp.concatenate(ms, axis=1).view(jnp.int32)
    idx = mk & jnp.int32(N - 1)
    val = (mk & jnp.int32(~(N - 1))).view(jnp.float32)
    val = jnp.where((val > 0.0) & (val < jnp.float32(1e-30)),
                    val / NEG_SCALE - NEG_OFF, val)

    dist_ref[0] = val
    src_ref[0] = idx + offset
    dst_ref[0] = jnp.broadcast_to(q_glob + offset, (QB, K))


def _bf16_round(v):
    bits = lax.bitcast_convert_type(v, jnp.int32)
    rbits = (bits + 0x7FFF + (lax.shift_right_logical(bits, 16) & 1)) & ~0xFFFF
    return lax.bitcast_convert_type(rbits, jnp.float32)


def _sc_knn(pts_ref, outd_ref, outs_ref, pf, pb, x2b, resd, ress):
    wid = lax.axis_index("s") * 2 + lax.axis_index("c")
    b = wid // WPB
    part = wid % WPB
    qstart = NT + part * QS

    pltpu.sync_copy(pts_ref.at[b], pf)

    lane = lax.iota(jnp.int32, L)

    def stage(c, _):
        sl = pl.ds(c * L, L)
        vx = pf[0, sl]
        vy = pf[1, sl]
        vz = pf[2, sl]
        pb[0, sl] = _bf16_round(vx)
        pb[1, sl] = _bf16_round(vy)
        pb[2, sl] = _bf16_round(vz)
        x2b[sl] = vx * vx + vy * vy + vz * vz
        return 0

    lax.fori_loop(0, N // L, stage, 0)

    def per_query(qi, _):
        q = qstart + qi
        qsl = pl.ds((q // L) * L, L)
        lsel = jnp.full((L,), q % L, jnp.int32)

        def bcast(v):
            return v.at[lsel].get(mode="promise_in_bounds")

        qx = bcast(pf[0, qsl])
        qy = bcast(pf[1, qsl])
        qz = bcast(pf[2, qsl])
        x2q = bcast(x2b[qsl])
        qxb = _bf16_round(qx)
        qyb = _bf16_round(qy)
        qzb = _bf16_round(qz)

        def chunk(c, state):
            sl = pl.ds(c * L, L)
            dots = qxb * pb[0, sl]
            dots = dots + qyb * pb[1, sl]
            dots = dots + qzb * pb[2, sl]
            dist = (x2q + x2b[sl]) - 2.0 * dots
            dk = jnp.where(dist < 0.0, (dist + NEG_OFF) * NEG_SCALE, dist)
            col = lane + c * L
            keys = lax.bitcast_convert_type(
                (lax.bitcast_convert_type(dk, jnp.int32) & IMASK) | col,
                jnp.float32)
            v = jnp.where(col == q, BIGF, keys)
            out = []
            for s in state:
                lo = jnp.minimum(s, v)
                v = jnp.maximum(s, v)
                out.append(lo)
            return tuple(out)

        init = tuple(jnp.full((L,), BIGF, jnp.float32) for _ in range(K))
        state = lax.fori_loop(0, N // L, chunk, init)

        cur = list(state)
        res = jnp.full((L,), BIGF, jnp.float32)
        for k in range(K):
            m = cur[0]
            for s in cur[1:]:
                m = jnp.minimum(m, s)
            for h in (8, 4, 2, 1):
                m = jnp.minimum(m, m.at[lane ^ h].get(mode="promise_in_bounds"))
            res = jnp.where(lane == k, m, res)
            cur = [jnp.where(s == m, BIGF, s) for s in cur]

        kb = lax.bitcast_convert_type(res, jnp.int32)
        idx = kb & (N - 1)
        vb = lax.bitcast_convert_type(kb & IMASK, jnp.float32)
        val = jnp.where((vb > 0.0) & (vb < 1e-30),
                        vb / NEG_SCALE - NEG_OFF, vb)
        osl = pl.ds(qi * K, K)
        resd[osl] = val
        ress[osl] = idx + b * N
        return 0

    lax.fori_loop(0, QS, per_query, 0)

    off = (b * NS + part * QS) * K
    pltpu.sync_copy(resd, outd_ref.at[pl.ds(off, QS * K)])
    pltpu.sync_copy(ress, outs_ref.at[pl.ds(off, QS * K)])


_sc_call = functools.partial(
    pl.kernel,
    out_type=[
        jax.ShapeDtypeStruct((B * NS * K,), jnp.float32),
        jax.ShapeDtypeStruct((B * NS * K,), jnp.int32),
    ],
    mesh=plsc.VectorSubcoreMesh(core_axis_name="c", subcore_axis_name="s"),
    scratch_types=[
        pltpu.VMEM((D, N), jnp.float32),
        pltpu.VMEM((D, N), jnp.float32),
        pltpu.VMEM((N,), jnp.float32),
        pltpu.VMEM((QS * K,), jnp.float32),
        pltpu.VMEM((QS * K,), jnp.int32),
    ],
)(_sc_knn)


@jax.jit
def kernel(points):
    pts_t = jnp.transpose(points, (0, 2, 1))
    grid = (B, NT // QB)
    dist_tc, src_tc, dst_tc = pl.pallas_call(
        _knn_tile,
        grid=grid,
        in_specs=[
            pl.BlockSpec((1, QB, D), lambda b, q: (b, q, 0)),
            pl.BlockSpec((1, D, N), lambda b, q: (b, 0, 0)),
        ],
        out_specs=[
            pl.BlockSpec((1, QB, K), lambda b, q: (b, q, 0)),
            pl.BlockSpec((1, QB, K), lambda b, q: (b, q, 0)),
            pl.BlockSpec((1, QB, K), lambda b, q: (b, q, 0)),
        ],
        out_shape=[
            jax.ShapeDtypeStruct((B, NT, K), jnp.float32),
            jax.ShapeDtypeStruct((B, NT, K), jnp.int32),
            jax.ShapeDtypeStruct((B, NT, K), jnp.int32),
        ],
    )(points, pts_t)

    dist_sc, src_sc = _sc_call(pts_t)
    dist_sc = dist_sc.reshape(B, NS, K)
    src_sc = src_sc.reshape(B, NS, K)
    dst_sc = (jnp.arange(NT, N, dtype=jnp.int32)[None, :, None]
              + (jnp.arange(B, dtype=jnp.int32) * N)[:, None, None])
    dst_sc = jnp.broadcast_to(dst_sc, (B, NS, K))

    dist = jnp.concatenate([dist_tc, dist_sc], axis=1)
    src = jnp.concatenate([src_tc, src_sc], axis=1)
    dst = jnp.concatenate([dst_tc, dst_sc], axis=1)
    edge_index = jnp.stack([src.reshape(-1), dst.reshape(-1)], axis=0)
    return (dist, edge_index)


# speedup vs baseline: 1.0205x; 1.0003x over previous
import functools

import jax
import jax.numpy as jnp
from jax import lax
from jax.experimental import pallas as pl
from jax.experimental.pallas import tpu as pltpu
from jax.experimental.pallas import tpu_sc as plsc

B = 8
N = 2048
D = 3
K = 16
QB = 512
NS = 512
NT = N - NS
NW = 32
WPB = NW // B
QS = NS // WPB
L = 16

NEG_SCALE = 1e-33
NEG_OFF = 0.125
BIGF = 3e38
IMASK = ~(N - 1)


def _knn_tile(q_ref, pt_ref, dist_ref, src_ref, dst_ref):
    b = pl.program_id(0)
    qb = pl.program_id(1)

    q = q_ref[0]
    p = pt_ref[0]

    dots = jax.lax.dot_general(
        q.astype(jnp.bfloat16), p.astype(jnp.bfloat16),
        (((1,), (0,)), ((), ())),
        preferred_element_type=jnp.float32,
    )
    x2q = jnp.sum(q * q, axis=1, keepdims=True)
    x2p = jnp.sum(p * p, axis=0, keepdims=True)
    dist = x2q + x2p - 2.0 * dots

    cols = jax.lax.broadcasted_iota(jnp.int32, (QB, N), 1)
    rows = jax.lax.broadcasted_iota(jnp.int32, (QB, 1), 0)
    q_glob = rows + qb * QB

    dk = jnp.where(dist < 0.0, (dist + NEG_OFF) * NEG_SCALE, dist)
    keys = ((dk.view(jnp.int32) & IMASK) | cols).view(jnp.float32)
    keys = jnp.where(cols == q_glob, BIGF, keys)

    offset = b * N
    ms = []
    for _ in range(K):
        m = jnp.min(keys, axis=1, keepdims=True)
        ms.append(m)
        keys = jnp.where(keys == m, BIGF, keys)

    mk = jnp.concatenate(ms, axis=1).view(jnp.int32)
    idx = mk & jnp.int32(N - 1)
    val = (mk & jnp.int32(~(N - 1))).view(jnp.float32)
    val = jnp.where((val > 0.0) & (val < jnp.float32(1e-30)),
                    val / NEG_SCALE - NEG_OFF, val)

    dist_ref[0] = val
    src_ref[0] = idx + offset
    dst_ref[0] = jnp.broadcast_to(q_glob + offset, (QB, K))


def _bf16_round(v):
    bits = lax.bitcast_convert_type(v, jnp.int32)
    rbits = (bits + 0x7FFF + (lax.shift_right_logical(bits, 16) & 1)) & ~0xFFFF
    return lax.bitcast_convert_type(rbits, jnp.float32)


def _sc_knn(pts_ref, outd_ref, outs_ref, pf, pb, x2b, resd, ress):
    wid = lax.axis_index("s") * 2 + lax.axis_index("c")
    b = wid // WPB
    part = wid % WPB
    qstart = NT + part * QS

    pltpu.sync_copy(pts_ref.at[b], pf)

    lane = lax.iota(jnp.int32, L)

    def stage(c, _):
        sl = pl.ds(c * L, L)
        vx = pf[0, sl]
        vy = pf[1, sl]
        vz = pf[2, sl]
        pb[0, sl] = _bf16_round(vx)
        pb[1, sl] = _bf16_round(vy)
        pb[2, sl] = _bf16_round(vz)
        x2b[sl] = vx * vx + vy * vy + vz * vz
        return 0

    lax.fori_loop(0, N // L, stage, 0)

    def per_query(qi, _):
        q = qstart + qi
        qsl = pl.ds((q // L) * L, L)
        lsel = jnp.full((L,), q % L, jnp.int32)

        def bcast(v):
            return v.at[lsel].get(mode="promise_in_bounds")

        qx = bcast(pf[0, qsl])
        qy = bcast(pf[1, qsl])
        qz = bcast(pf[2, qsl])
        x2q = bcast(x2b[qsl])
        qxb = _bf16_round(qx)
        qyb = _bf16_round(qy)
        qzb = _bf16_round(qz)

        def chunk(c, state):
            sl = pl.ds(c * L, L)
            dots = qxb * pb[0, sl]
            dots = dots + qyb * pb[1, sl]
            dots = dots + qzb * pb[2, sl]
            dist = (x2q + x2b[sl]) - 2.0 * dots
            dk = jnp.where(dist < 0.0, (dist + NEG_OFF) * NEG_SCALE, dist)
            col = lane + c * L
            keys = lax.bitcast_convert_type(
                (lax.bitcast_convert_type(dk, jnp.int32) & IMASK) | col,
                jnp.float32)
            v = jnp.where(col == q, BIGF, keys)
            out = []
            for s in state:
                lo = jnp.minimum(s, v)
                v = jnp.maximum(s, v)
                out.append(lo)
            return tuple(out)

        init = tuple(jnp.full((L,), BIGF, jnp.float32) for _ in range(K))
        state = lax.fori_loop(0, N // L, chunk, init)

        cur = list(state)
        res = jnp.full((L,), BIGF, jnp.float32)
        for k in range(K):
            m = cur[0]
            for s in cur[1:]:
                m = jnp.minimum(m, s)
            for h in (8, 4, 2, 1):
                m = jnp.minimum(m, m.at[lane ^ h].get(mode="promise_in_bounds"))
            res = jnp.where(lane == k, m, res)
            cur = [jnp.where(s == m, BIGF, s) for s in cur]

        kb = lax.bitcast_convert_type(res, jnp.int32)
        idx = kb & (N - 1)
        vb = lax.bitcast_convert_type(kb & IMASK, jnp.float32)
        val = jnp.where((vb > 0.0) & (vb < 1e-30),
                        vb / NEG_SCALE - NEG_OFF, vb)
        osl = pl.ds(qi * K, K)
        resd[osl] = val
        ress[osl] = idx + b * N
        return 0

    lax.fori_loop(0, QS, per_query, 0)

    off = (b * NS + part * QS) * K
    pltpu.sync_copy(resd, outd_ref.at[pl.ds(off, QS * K)])
    pltpu.sync_copy(ress, outs_ref.at[pl.ds(off, QS * K)])


_sc_call = functools.partial(
    pl.kernel,
    out_type=[
        jax.ShapeDtypeStruct((B * NS * K,), jnp.float32),
        jax.ShapeDtypeStruct((B * NS * K,), jnp.int32),
    ],
    mesh=plsc.VectorSubcoreMesh(core_axis_name="c", subcore_axis_name="s"),
    scratch_types=[
        pltpu.VMEM((D, N), jnp.float32),
        pltpu.VMEM((D, N), jnp.float32),
        pltpu.VMEM((N,), jnp.float32),
        pltpu.VMEM((QS * K,), jnp.float32),
        pltpu.VMEM((QS * K,), jnp.int32),
    ],
)(_sc_knn)


@jax.jit
def kernel(points):
    pts_t = jnp.transpose(points, (0, 2, 1))
    dist_sc, src_sc = _sc_call(pts_t)
    grid = (B, NT // QB)
    dist_tc, src_tc, dst_tc = pl.pallas_call(
        _knn_tile,
        grid=grid,
        in_specs=[
            pl.BlockSpec((1, QB, D), lambda b, q: (b, q, 0)),
            pl.BlockSpec((1, D, N), lambda b, q: (b, 0, 0)),
        ],
        out_specs=[
            pl.BlockSpec((1, QB, K), lambda b, q: (b, q, 0)),
            pl.BlockSpec((1, QB, K), lambda b, q: (b, q, 0)),
            pl.BlockSpec((1, QB, K), lambda b, q: (b, q, 0)),
        ],
        out_shape=[
            jax.ShapeDtypeStruct((B, NT, K), jnp.float32),
            jax.ShapeDtypeStruct((B, NT, K), jnp.int32),
            jax.ShapeDtypeStruct((B, NT, K), jnp.int32),
        ],
    )(points, pts_t)

    dist_sc = dist_sc.reshape(B, NS, K)
    src_sc = src_sc.reshape(B, NS, K)
    dst_sc = (jnp.arange(NT, N, dtype=jnp.int32)[None, :, None]
              + (jnp.arange(B, dtype=jnp.int32) * N)[:, None, None])
    dst_sc = jnp.broadcast_to(dst_sc, (B, NS, K))

    dist = jnp.concatenate([dist_tc, dist_sc], axis=1)
    src = jnp.concatenate([src_tc, src_sc], axis=1)
    dst = jnp.concatenate([dst_tc, dst_sc], axis=1)
    edge_index = jnp.stack([src.reshape(-1), dst.reshape(-1)], axis=0)
    return (dist, edge_index)
